# baseline (speedup 1.0000x reference)
import functools

import jax
import jax.numpy as jnp
from jax import lax
from jax.experimental import pallas as pl
from jax.experimental.pallas import tpu as pltpu
from jax.experimental.pallas import tpu_sc as plsc

N_C = 10000
N_G = 10000
D = 128
ALPHA1 = 0.5

NS = 16
B = 128
CH = 32
NB_MAIN = 160
NB_GG = 80
NB_MAIN_HALF = 80
NB_GG_HALF = 40
ACC_MAIN_ROWS = 10240


def _zero_stripe(zeros, buf, acc, off, n):
    pltpu.sync_copy(zeros, buf)
    o = 0
    while n > 0:
        c = min(n, B)
        pltpu.sync_copy(buf.at[pl.ds(0, c)], acc.at[pl.ds(off + o, c)])
        n -= c
        o += c


def _run_relation(tab_hbm, g_hbm, s_hbm, sub, b0, nb, acc,
                  idx_g_c, idx_s_c, rows_v, sg_a, ss_a, sg_b, ss_b, ch=CH):
    @pl.loop(0, nb, step=ch)
    def _(c):
        pltpu.sync_copy(g_hbm.at[sub, pl.ds(b0 + c, ch)],
                        idx_g_c.at[pl.ds(0, ch)])
        pltpu.sync_copy(s_hbm.at[sub, pl.ds(b0 + c, ch)],
                        idx_s_c.at[pl.ds(0, ch)])

        pltpu.async_copy(tab_hbm.at[idx_g_c.at[0]], rows_v.at[0], sg_a)

        @pl.loop(0, ch, step=2)
        def _(j):
            pltpu.make_async_copy(tab_hbm.at[idx_g_c.at[j]], rows_v.at[0],
                                  sg_a).wait()
            pltpu.async_copy(tab_hbm.at[idx_g_c.at[j + 1]], rows_v.at[1],
                             sg_b)
            pltpu.sync_copy(rows_v.at[0], acc.at[idx_s_c.at[j]], add=True)
            pltpu.make_async_copy(tab_hbm.at[idx_g_c.at[j + 1]],
                                  rows_v.at[1], sg_b).wait()

            @pl.when(j + 2 < ch)
            def _():
                pltpu.async_copy(tab_hbm.at[idx_g_c.at[j + 2]], rows_v.at[0],
                                 sg_a)

            pltpu.sync_copy(rows_v.at[1], acc.at[idx_s_c.at[j + 1]],
                            add=True)


_SC_SCRATCH = [
    pltpu.VMEM((CH, B), jnp.int32),
    pltpu.VMEM((CH, B), jnp.int32),
    pltpu.VMEM((2, B, D), jnp.float32),
    pltpu.SemaphoreType.DMA,
    pltpu.SemaphoreType.DMA,
    pltpu.SemaphoreType.DMA,
    pltpu.SemaphoreType.DMA,
]


def _fused_body(w_c, w_g, w_gg, rev_g, rev_s, exp_g, exp_s, gg_g, gg_t,
                zeros, oc_a, oc_b, oge_a, oge_b, ogg_a, ogg_b,
                acc, idx_g_c, idx_s_c, rows_v, sg_a, ss_a, sg_b, ss_b):
    core = lax.axis_index("core")
    sub = lax.axis_index("subcore")
    stripe = ACC_MAIN_ROWS // NS
    off = sub * stripe

    def phase(tab, g_idx, s_idx, nb_half, ch, out_a, out_b, zero_after):
        _run_relation(tab, g_idx, s_idx, sub, core * nb_half, nb_half, acc,
                      idx_g_c, idx_s_c, rows_v, sg_a, ss_a, sg_b, ss_b,
                      ch=ch)
        plsc.subcore_barrier()

        @pl.when(core == 0)
        def _():
            pltpu.sync_copy(acc.at[pl.ds(off, stripe)],
                            out_a.at[pl.ds(off, stripe)])

        @pl.when(core == 1)
        def _():
            pltpu.sync_copy(acc.at[pl.ds(off, stripe)],
                            out_b.at[pl.ds(off, stripe)])

        if zero_after:
            _zero_stripe(zeros, rows_v.at[0], acc, off, stripe)
            plsc.subcore_barrier()

    _zero_stripe(zeros, rows_v.at[0], acc, off, stripe)
    plsc.subcore_barrier()
    phase(w_g, rev_g, rev_s, NB_MAIN_HALF, 16, oc_a, oc_b, zero_after=True)
    phase(w_c, exp_g, exp_s, NB_MAIN_HALF, 16, oge_a, oge_b, zero_after=True)
    phase(w_gg, gg_g, gg_t, NB_GG_HALF, 8, ogg_a, ogg_b, zero_after=False)


def _sc_mesh():
    return plsc.VectorSubcoreMesh(core_axis_name="core",
                                  subcore_axis_name="subcore",
                                  num_cores=2, num_subcores=NS)


@functools.cache
def _fused_kernel():
    return pl.kernel(
        _fused_body,
        out_type=tuple(
            jax.ShapeDtypeStruct((ACC_MAIN_ROWS, D), jnp.float32)
            for _ in range(6)
        ),
        mesh=_sc_mesh(),
        scratch_types=[pltpu.VMEM_SHARED((ACC_MAIN_ROWS, D), jnp.float32)]
        + _SC_SCRATCH,
    )


def _prescale_body(c_ref, g_ref, cjc_ref, cjg_ref, cjj_ref,
                   wc_ref, wg_ref, wgg_ref):
    wc_ref[...] = c_ref[...] * cjc_ref[...]
    wg_ref[...] = g_ref[...] * cjg_ref[...]
    wgg_ref[...] = g_ref[...] * cjj_ref[...]


def _prescale(c_feat, g_feat, cj_cell, cj_gene, cjj_gene):
    blk = 1000
    grid = N_C // blk
    feat_spec = pl.BlockSpec((blk, D), lambda i: (i, 0))
    scale_spec = pl.BlockSpec((blk, 1), lambda i: (i, 0))
    return pl.pallas_call(
        _prescale_body,
        grid=(grid,),
        in_specs=[feat_spec, feat_spec, scale_spec, scale_spec, scale_spec],
        out_specs=[feat_spec, feat_spec, feat_spec],
        out_shape=[
            jax.ShapeDtypeStruct((N_C, D), jnp.float32),
            jax.ShapeDtypeStruct((N_G, D), jnp.float32),
            jax.ShapeDtypeStruct((N_G, D), jnp.float32),
        ],
    )(c_feat, g_feat, cj_cell, cj_gene, cjj_gene)


def _postscale_body(oca_ref, ocb_ref, gea_ref, geb_ref, ga_ref, gb_ref,
                    cic_ref, cig_ref, cii_ref, out_c_ref, g_out_ref):
    out_c_ref[...] = (oca_ref[...] + ocb_ref[...]) * cic_ref[...]
    g_out_ref[...] = \
        ALPHA1 * ((gea_ref[...] + geb_ref[...]) * cig_ref[...]) + \
        (1.0 - ALPHA1) * ((ga_ref[...] + gb_ref[...]) * cii_ref[...])


def _postscale(oc_a, oc_b, oge_a, oge_b, ogg_a, ogg_b,
               ci_cell, ci_gene, cii_gene):
    blk = 1000
    grid = N_G // blk
    full_spec = pl.BlockSpec((blk, D), lambda i: (i, 0))
    scale_spec = pl.BlockSpec((blk, 1), lambda i: (i, 0))
    return pl.pallas_call(
        _postscale_body,
        grid=(grid,),
        in_specs=[full_spec] * 6 + [scale_spec] * 3,
        out_specs=[full_spec, full_spec],
        out_shape=[
            jax.ShapeDtypeStruct((N_C, D), jnp.float32),
            jax.ShapeDtypeStruct((N_G, D), jnp.float32),
        ],
    )(oc_a, oc_b, oge_a, oge_b, ogg_a, ogg_b, ci_cell, ci_gene, cii_gene)


def _pad_reshape(idx, total, pad_val, nt, nb):
    pad = total - idx.shape[0]
    idx = jnp.concatenate(
        [idx, jnp.full((pad,), pad_val, dtype=jnp.int32)])
    return idx.reshape(nt, nb, B)


def kernel(c_feat, g_feat, cj_cell, ci_cell, cj_gene, ci_gene, cjj_gene,
           cii_gene, exp_edge_index, gg_edge_index):
    exp_src = exp_edge_index[0].astype(jnp.int32)
    exp_dst = exp_edge_index[1].astype(jnp.int32)
    gg_s = gg_edge_index[0].astype(jnp.int32)
    gg_t = gg_edge_index[1].astype(jnp.int32)

    tot_main = NS * NB_MAIN * B
    tot_gg = NS * NB_GG * B
    rev_g = _pad_reshape(exp_dst, tot_main, 0, NS, NB_MAIN)
    rev_s = _pad_reshape(exp_src, tot_main, N_C, NS, NB_MAIN)
    exp_g = _pad_reshape(exp_src, tot_main, 0, NS, NB_MAIN)
    exp_s = _pad_reshape(exp_dst, tot_main, N_G, NS, NB_MAIN)
    gg_g = _pad_reshape(gg_s, tot_gg, 0, NS, NB_GG)
    gg_t2 = _pad_reshape(gg_t, tot_gg, N_G, NS, NB_GG)

    w_c, w_g, w_gg = _prescale(c_feat, g_feat, cj_cell, cj_gene, cjj_gene)
    zeros = jnp.zeros((B, D), jnp.float32)

    oc_a, oc_b, oge_a, oge_b, ogg_a, ogg_b = _fused_kernel()(
        w_c, w_g, w_gg, rev_g, rev_s, exp_g, exp_s, gg_g, gg_t2, zeros)

    out_c, g_out = _postscale(oc_a, oc_b, oge_a, oge_b, ogg_a, ogg_b,
                              ci_cell, ci_gene, cii_gene)
    return (out_c, g_out)


# speedup vs baseline: 1.3964x; 1.3964x over previous
import functools

import jax
import jax.numpy as jnp
from jax import lax
from jax.experimental import pallas as pl
from jax.experimental.pallas import tpu as pltpu
from jax.experimental.pallas import tpu_sc as plsc

N_C = 10000
N_G = 10000
D = 128
ALPHA1 = 0.5

NS = 16
B = 128
CH = 32
NB_MAIN = 160
NB_GG = 80
GG_B0 = 40
ACC_MAIN_ROWS = 10240


def _zero_stripe(zeros, buf, acc, off, n):
    pltpu.sync_copy(zeros, buf)
    o = 0
    while n > 0:
        c = min(n, B)
        pltpu.sync_copy(buf.at[pl.ds(0, c)], acc.at[pl.ds(off + o, c)])
        n -= c
        o += c


def _run_relation(tab_hbm, g_hbm, s_hbm, sub, b0, nb, acc,
                  idx_g_c, idx_s_c, rows_v, sg_a, ss_a, sg_b, ss_b, ch=CH):
    @pl.loop(0, nb, step=ch)
    def _(c):
        pltpu.sync_copy(g_hbm.at[sub, pl.ds(b0 + c, ch)],
                        idx_g_c.at[pl.ds(0, ch)])
        pltpu.sync_copy(s_hbm.at[sub, pl.ds(b0 + c, ch)],
                        idx_s_c.at[pl.ds(0, ch)])

        pltpu.async_copy(tab_hbm.at[idx_g_c.at[0]], rows_v.at[0], sg_a)

        @pl.loop(0, ch, step=2)
        def _(j):
            pltpu.make_async_copy(tab_hbm.at[idx_g_c.at[j]], rows_v.at[0],
                                  sg_a).wait()
            pltpu.async_copy(tab_hbm.at[idx_g_c.at[j + 1]], rows_v.at[1],
                             sg_b)
            pltpu.sync_copy(rows_v.at[0], acc.at[idx_s_c.at[j]], add=True)
            pltpu.make_async_copy(tab_hbm.at[idx_g_c.at[j + 1]],
                                  rows_v.at[1], sg_b).wait()

            @pl.when(j + 2 < ch)
            def _():
                pltpu.async_copy(tab_hbm.at[idx_g_c.at[j + 2]], rows_v.at[0],
                                 sg_a)

            pltpu.sync_copy(rows_v.at[1], acc.at[idx_s_c.at[j + 1]],
                            add=True)


_SC_SCRATCH = [
    pltpu.VMEM((CH, B), jnp.int32),
    pltpu.VMEM((CH, B), jnp.int32),
    pltpu.VMEM((2, B, D), jnp.float32),
    pltpu.SemaphoreType.DMA,
    pltpu.SemaphoreType.DMA,
    pltpu.SemaphoreType.DMA,
    pltpu.SemaphoreType.DMA,
]


def _fused_body(w_c, w_g, w_gg, rev_g, rev_s, exp_g, exp_s, gg_g, gg_t,
                zeros, out_c_raw, out_ge_raw, out_gg_a, out_gg_b,
                acc, idx_g_c, idx_s_c, rows_v, sg_a, ss_a, sg_b, ss_b):
    core = lax.axis_index("core")
    sub = lax.axis_index("subcore")
    stripe = ACC_MAIN_ROWS // NS
    off = sub * stripe

    _zero_stripe(zeros, rows_v.at[0], acc, off, stripe)
    plsc.subcore_barrier()

    @pl.when(core == 0)
    def _():
        _run_relation(w_g, rev_g, rev_s, sub, 0, NB_MAIN, acc,
                      idx_g_c, idx_s_c, rows_v, sg_a, ss_a, sg_b, ss_b)

    @pl.when(core == 1)
    def _():
        _run_relation(w_c, exp_g, exp_s, sub, 0, NB_MAIN, acc,
                      idx_g_c, idx_s_c, rows_v, sg_a, ss_a, sg_b, ss_b)

    plsc.subcore_barrier()

    @pl.when(core == 0)
    def _():
        pltpu.sync_copy(acc.at[pl.ds(off, stripe)],
                        out_c_raw.at[pl.ds(off, stripe)])

    @pl.when(core == 1)
    def _():
        pltpu.sync_copy(acc.at[pl.ds(off, stripe)],
                        out_ge_raw.at[pl.ds(off, stripe)])

    _zero_stripe(zeros, rows_v.at[0], acc, off, stripe)
    plsc.subcore_barrier()

    @pl.when(core == 0)
    def _():
        _run_relation(w_gg, gg_g, gg_t, sub, 0, GG_B0, acc,
                      idx_g_c, idx_s_c, rows_v, sg_a, ss_a, sg_b, ss_b,
                      ch=8)

    @pl.when(core == 1)
    def _():
        _run_relation(w_gg, gg_g, gg_t, sub, GG_B0, NB_GG - GG_B0, acc,
                      idx_g_c, idx_s_c, rows_v, sg_a, ss_a, sg_b, ss_b,
                      ch=8)

    plsc.subcore_barrier()

    @pl.when(core == 0)
    def _():
        pltpu.sync_copy(acc.at[pl.ds(off, stripe)],
                        out_gg_a.at[pl.ds(off, stripe)])

    @pl.when(core == 1)
    def _():
        pltpu.sync_copy(acc.at[pl.ds(off, stripe)],
                        out_gg_b.at[pl.ds(off, stripe)])


def _sc_mesh():
    return plsc.VectorSubcoreMesh(core_axis_name="core",
                                  subcore_axis_name="subcore",
                                  num_cores=2, num_subcores=NS)


@functools.cache
def _fused_kernel():
    return pl.kernel(
        _fused_body,
        out_type=tuple(
            jax.ShapeDtypeStruct((ACC_MAIN_ROWS, D), jnp.float32)
            for _ in range(4)
        ),
        mesh=_sc_mesh(),
        scratch_types=[pltpu.VMEM_SHARED((ACC_MAIN_ROWS, D), jnp.float32)]
        + _SC_SCRATCH,
    )


def _prescale_body(c_ref, g_ref, cjc_ref, cjg_ref, cjj_ref,
                   wc_ref, wg_ref, wgg_ref):
    wc_ref[...] = c_ref[...] * cjc_ref[...]
    wg_ref[...] = g_ref[...] * cjg_ref[...]
    wgg_ref[...] = g_ref[...] * cjj_ref[...]


def _prescale(c_feat, g_feat, cj_cell, cj_gene, cjj_gene):
    blk = 1000
    grid = N_C // blk
    feat_spec = pl.BlockSpec((blk, D), lambda i: (i, 0))
    scale_spec = pl.BlockSpec((blk, 1), lambda i: (i, 0))
    return pl.pallas_call(
        _prescale_body,
        grid=(grid,),
        in_specs=[feat_spec, feat_spec, scale_spec, scale_spec, scale_spec],
        out_specs=[feat_spec, feat_spec, feat_spec],
        out_shape=[
            jax.ShapeDtypeStruct((N_C, D), jnp.float32),
            jax.ShapeDtypeStruct((N_G, D), jnp.float32),
            jax.ShapeDtypeStruct((N_G, D), jnp.float32),
        ],
    )(c_feat, g_feat, cj_cell, cj_gene, cjj_gene)


def _postscale_body(oc_ref, oge_ref, ga_ref, gb_ref, cic_ref, cig_ref,
                    cii_ref, out_c_ref, g_out_ref):
    out_c_ref[...] = oc_ref[...] * cic_ref[...]
    g_out_ref[...] = ALPHA1 * (oge_ref[...] * cig_ref[...]) + \
        (1.0 - ALPHA1) * ((ga_ref[...] + gb_ref[...]) * cii_ref[...])


def _postscale(oc_raw, oge_raw, ogg_a, ogg_b, ci_cell, ci_gene, cii_gene):
    blk = 1000
    grid = N_G // blk
    full_spec = pl.BlockSpec((blk, D), lambda i: (i, 0))
    scale_spec = pl.BlockSpec((blk, 1), lambda i: (i, 0))
    return pl.pallas_call(
        _postscale_body,
        grid=(grid,),
        in_specs=[full_spec] * 4 + [scale_spec] * 3,
        out_specs=[full_spec, full_spec],
        out_shape=[
            jax.ShapeDtypeStruct((N_C, D), jnp.float32),
            jax.ShapeDtypeStruct((N_G, D), jnp.float32),
        ],
    )(oc_raw, oge_raw, ogg_a, ogg_b, ci_cell, ci_gene, cii_gene)


def _pad_reshape(idx, total, pad_val, nt, nb):
    pad = total - idx.shape[0]
    idx = jnp.concatenate(
        [idx, jnp.full((pad,), pad_val, dtype=jnp.int32)])
    return idx.reshape(nt, nb, B)


def kernel(c_feat, g_feat, cj_cell, ci_cell, cj_gene, ci_gene, cjj_gene,
           cii_gene, exp_edge_index, gg_edge_index):
    exp_src = exp_edge_index[0].astype(jnp.int32)
    exp_dst = exp_edge_index[1].astype(jnp.int32)
    gg_s = gg_edge_index[0].astype(jnp.int32)
    gg_t = gg_edge_index[1].astype(jnp.int32)

    tot_main = NS * NB_MAIN * B
    tot_gg = NS * NB_GG * B
    rev_g = _pad_reshape(exp_dst, tot_main, 0, NS, NB_MAIN)
    rev_s = _pad_reshape(exp_src, tot_main, N_C, NS, NB_MAIN)
    exp_g = _pad_reshape(exp_src, tot_main, 0, NS, NB_MAIN)
    exp_s = _pad_reshape(exp_dst, tot_main, N_G, NS, NB_MAIN)
    gg_g = _pad_reshape(gg_s, tot_gg, 0, NS, NB_GG)
    gg_t2 = _pad_reshape(gg_t, tot_gg, N_G, NS, NB_GG)

    w_c, w_g, w_gg = _prescale(c_feat, g_feat, cj_cell, cj_gene, cjj_gene)
    zeros = jnp.zeros((B, D), jnp.float32)

    oc_raw, oge_raw, ogg_a, ogg_b = _fused_kernel()(
        w_c, w_g, w_gg, rev_g, rev_s, exp_g, exp_s, gg_g, gg_t2, zeros)

    out_c, g_out = _postscale(oc_raw, oge_raw, ogg_a, ogg_b,
                              ci_cell, ci_gene, cii_gene)
    return (out_c, g_out)


# speedup vs baseline: 1.4140x; 1.0126x over previous
import functools

import jax
import jax.numpy as jnp
from jax import lax
from jax.experimental import pallas as pl
from jax.experimental.pallas import tpu as pltpu
from jax.experimental.pallas import tpu_sc as plsc

N_C = 10000
N_G = 10000
D = 128
ALPHA1 = 0.5

NS = 16
B = 128
CH = 32
NB_MAIN = 160
NB_GG = 80
GG_B0 = 48
ACC_MAIN_ROWS = 10240


def _zero_stripe(zeros, buf, acc, off, n):
    pltpu.sync_copy(zeros, buf)
    o = 0
    while n > 0:
        c = min(n, B)
        pltpu.sync_copy(buf.at[pl.ds(0, c)], acc.at[pl.ds(off + o, c)])
        n -= c
        o += c


def _run_relation(tab_hbm, g_hbm, s_hbm, sub, b0, nb, acc,
                  idx_g_c, idx_s_c, rows_v, sg_a, ss_a, sg_b, ss_b, ch=CH):
    @pl.loop(0, nb, step=ch)
    def _(c):
        pltpu.sync_copy(g_hbm.at[sub, pl.ds(b0 + c, ch)],
                        idx_g_c.at[pl.ds(0, ch)])
        pltpu.sync_copy(s_hbm.at[sub, pl.ds(b0 + c, ch)],
                        idx_s_c.at[pl.ds(0, ch)])

        pltpu.async_copy(tab_hbm.at[idx_g_c.at[0]], rows_v.at[0], sg_a)

        @pl.loop(0, ch, step=2)
        def _(j):
            pltpu.make_async_copy(tab_hbm.at[idx_g_c.at[j]], rows_v.at[0],
                                  sg_a).wait()
            pltpu.async_copy(tab_hbm.at[idx_g_c.at[j + 1]], rows_v.at[1],
                             sg_b)
            pltpu.sync_copy(rows_v.at[0], acc.at[idx_s_c.at[j]], add=True)
            pltpu.make_async_copy(tab_hbm.at[idx_g_c.at[j + 1]],
                                  rows_v.at[1], sg_b).wait()

            @pl.when(j + 2 < ch)
            def _():
                pltpu.async_copy(tab_hbm.at[idx_g_c.at[j + 2]], rows_v.at[0],
                                 sg_a)

            pltpu.sync_copy(rows_v.at[1], acc.at[idx_s_c.at[j + 1]],
                            add=True)


_SC_SCRATCH = [
    pltpu.VMEM((CH, B), jnp.int32),
    pltpu.VMEM((CH, B), jnp.int32),
    pltpu.VMEM((2, B, D), jnp.float32),
    pltpu.SemaphoreType.DMA,
    pltpu.SemaphoreType.DMA,
    pltpu.SemaphoreType.DMA,
    pltpu.SemaphoreType.DMA,
]


def _fused_body(w_c, w_g, w_gg, rev_g, rev_s, exp_g, exp_s, gg_g, gg_t,
                zeros, out_c_raw, out_ge_raw, out_gg_a, out_gg_b,
                acc, idx_g_c, idx_s_c, rows_v, sg_a, ss_a, sg_b, ss_b):
    core = lax.axis_index("core")
    sub = lax.axis_index("subcore")
    stripe = ACC_MAIN_ROWS // NS
    off = sub * stripe

    _zero_stripe(zeros, rows_v.at[0], acc, off, stripe)
    plsc.subcore_barrier()

    @pl.when(core == 0)
    def _():
        _run_relation(w_g, rev_g, rev_s, sub, 0, NB_MAIN, acc,
                      idx_g_c, idx_s_c, rows_v, sg_a, ss_a, sg_b, ss_b)

    @pl.when(core == 1)
    def _():
        _run_relation(w_c, exp_g, exp_s, sub, 0, NB_MAIN, acc,
                      idx_g_c, idx_s_c, rows_v, sg_a, ss_a, sg_b, ss_b)

    plsc.subcore_barrier()

    @pl.when(core == 0)
    def _():
        pltpu.sync_copy(acc.at[pl.ds(off, stripe)],
                        out_c_raw.at[pl.ds(off, stripe)])

    @pl.when(core == 1)
    def _():
        pltpu.sync_copy(acc.at[pl.ds(off, stripe)],
                        out_ge_raw.at[pl.ds(off, stripe)])

    _zero_stripe(zeros, rows_v.at[0], acc, off, stripe)
    plsc.subcore_barrier()

    @pl.when(core == 0)
    def _():
        _run_relation(w_gg, gg_g, gg_t, sub, 0, GG_B0, acc,
                      idx_g_c, idx_s_c, rows_v, sg_a, ss_a, sg_b, ss_b,
                      ch=8)

    @pl.when(core == 1)
    def _():
        _run_relation(w_gg, gg_g, gg_t, sub, GG_B0, NB_GG - GG_B0, acc,
                      idx_g_c, idx_s_c, rows_v, sg_a, ss_a, sg_b, ss_b,
                      ch=8)

    plsc.subcore_barrier()

    @pl.when(core == 0)
    def _():
        pltpu.sync_copy(acc.at[pl.ds(off, stripe)],
                        out_gg_a.at[pl.ds(off, stripe)])

    @pl.when(core == 1)
    def _():
        pltpu.sync_copy(acc.at[pl.ds(off, stripe)],
                        out_gg_b.at[pl.ds(off, stripe)])


def _sc_mesh():
    return plsc.VectorSubcoreMesh(core_axis_name="core",
                                  subcore_axis_name="subcore",
                                  num_cores=2, num_subcores=NS)


@functools.cache
def _fused_kernel():
    return pl.kernel(
        _fused_body,
        out_type=tuple(
            jax.ShapeDtypeStruct((ACC_MAIN_ROWS, D), jnp.float32)
            for _ in range(4)
        ),
        mesh=_sc_mesh(),
        scratch_types=[pltpu.VMEM_SHARED((ACC_MAIN_ROWS, D), jnp.float32)]
        + _SC_SCRATCH,
    )


def _prescale_body(c_ref, g_ref, cjc_ref, cjg_ref, cjj_ref,
                   wc_ref, wg_ref, wgg_ref):
    wc_ref[...] = c_ref[...] * cjc_ref[...]
    wg_ref[...] = g_ref[...] * cjg_ref[...]
    wgg_ref[...] = g_ref[...] * cjj_ref[...]


def _prescale(c_feat, g_feat, cj_cell, cj_gene, cjj_gene):
    blk = 1000
    grid = N_C // blk
    feat_spec = pl.BlockSpec((blk, D), lambda i: (i, 0))
    scale_spec = pl.BlockSpec((blk, 1), lambda i: (i, 0))
    return pl.pallas_call(
        _prescale_body,
        grid=(grid,),
        in_specs=[feat_spec, feat_spec, scale_spec, scale_spec, scale_spec],
        out_specs=[feat_spec, feat_spec, feat_spec],
        out_shape=[
            jax.ShapeDtypeStruct((N_C, D), jnp.float32),
            jax.ShapeDtypeStruct((N_G, D), jnp.float32),
            jax.ShapeDtypeStruct((N_G, D), jnp.float32),
        ],
    )(c_feat, g_feat, cj_cell, cj_gene, cjj_gene)


def _postscale_body(oc_ref, oge_ref, ga_ref, gb_ref, cic_ref, cig_ref,
                    cii_ref, out_c_ref, g_out_ref):
    out_c_ref[...] = oc_ref[...] * cic_ref[...]
    g_out_ref[...] = ALPHA1 * (oge_ref[...] * cig_ref[...]) + \
        (1.0 - ALPHA1) * ((ga_ref[...] + gb_ref[...]) * cii_ref[...])


def _postscale(oc_raw, oge_raw, ogg_a, ogg_b, ci_cell, ci_gene, cii_gene):
    blk = 1000
    grid = N_G // blk
    full_spec = pl.BlockSpec((blk, D), lambda i: (i, 0))
    scale_spec = pl.BlockSpec((blk, 1), lambda i: (i, 0))
    return pl.pallas_call(
        _postscale_body,
        grid=(grid,),
        in_specs=[full_spec] * 4 + [scale_spec] * 3,
        out_specs=[full_spec, full_spec],
        out_shape=[
            jax.ShapeDtypeStruct((N_C, D), jnp.float32),
            jax.ShapeDtypeStruct((N_G, D), jnp.float32),
        ],
    )(oc_raw, oge_raw, ogg_a, ogg_b, ci_cell, ci_gene, cii_gene)


def _pad_reshape(idx, total, pad_val, nt, nb):
    pad = total - idx.shape[0]
    idx = jnp.concatenate(
        [idx, jnp.full((pad,), pad_val, dtype=jnp.int32)])
    return idx.reshape(nt, nb, B)


def kernel(c_feat, g_feat, cj_cell, ci_cell, cj_gene, ci_gene, cjj_gene,
           cii_gene, exp_edge_index, gg_edge_index):
    exp_src = exp_edge_index[0].astype(jnp.int32)
    exp_dst = exp_edge_index[1].astype(jnp.int32)
    gg_s = gg_edge_index[0].astype(jnp.int32)
    gg_t = gg_edge_index[1].astype(jnp.int32)

    tot_main = NS * NB_MAIN * B
    tot_gg = NS * NB_GG * B
    rev_g = _pad_reshape(exp_dst, tot_main, 0, NS, NB_MAIN)
    rev_s = _pad_reshape(exp_src, tot_main, N_C, NS, NB_MAIN)
    exp_g = _pad_reshape(exp_src, tot_main, 0, NS, NB_MAIN)
    exp_s = _pad_reshape(exp_dst, tot_main, N_G, NS, NB_MAIN)
    gg_g = _pad_reshape(gg_s, tot_gg, 0, NS, NB_GG)
    gg_t2 = _pad_reshape(gg_t, tot_gg, N_G, NS, NB_GG)

    w_c, w_g, w_gg = _prescale(c_feat, g_feat, cj_cell, cj_gene, cjj_gene)
    zeros = jnp.zeros((B, D), jnp.float32)

    oc_raw, oge_raw, ogg_a, ogg_b = _fused_kernel()(
        w_c, w_g, w_gg, rev_g, rev_s, exp_g, exp_s, gg_g, gg_t2, zeros)

    out_c, g_out = _postscale(oc_raw, oge_raw, ogg_a, ogg_b,
                              ci_cell, ci_gene, cii_gene)
    return (out_c, g_out)


# speedup vs baseline: 1.4629x; 1.0346x over previous
import functools

import jax
import jax.numpy as jnp
from jax import lax
from jax.experimental import pallas as pl
from jax.experimental.pallas import tpu as pltpu
from jax.experimental.pallas import tpu_sc as plsc

N_C = 10000
N_G = 10000
D = 128
ALPHA1 = 0.5

NS = 16
B = 128
CH = 32
NB_MAIN = 160
NB_GG = 80
GG_B0 = 56
ACC_MAIN_ROWS = 10240


def _zero_stripe(zeros, buf, acc, off, n):
    pltpu.sync_copy(zeros, buf)
    o = 0
    while n > 0:
        c = min(n, B)
        pltpu.sync_copy(buf.at[pl.ds(0, c)], acc.at[pl.ds(off + o, c)])
        n -= c
        o += c


def _run_relation(tab_hbm, g_hbm, s_hbm, sub, b0, nb, acc,
                  idx_g_c, idx_s_c, rows_v, sg_a, ss_a, sg_b, ss_b, ch=CH):
    @pl.loop(0, nb, step=ch)
    def _(c):
        pltpu.sync_copy(g_hbm.at[sub, pl.ds(b0 + c, ch)],
                        idx_g_c.at[pl.ds(0, ch)])
        pltpu.sync_copy(s_hbm.at[sub, pl.ds(b0 + c, ch)],
                        idx_s_c.at[pl.ds(0, ch)])

        pltpu.async_copy(tab_hbm.at[idx_g_c.at[0]], rows_v.at[0], sg_a)

        @pl.loop(0, ch, step=2)
        def _(j):
            pltpu.make_async_copy(tab_hbm.at[idx_g_c.at[j]], rows_v.at[0],
                                  sg_a).wait()
            pltpu.async_copy(tab_hbm.at[idx_g_c.at[j + 1]], rows_v.at[1],
                             sg_b)
            pltpu.sync_copy(rows_v.at[0], acc.at[idx_s_c.at[j]], add=True)
            pltpu.make_async_copy(tab_hbm.at[idx_g_c.at[j + 1]],
                                  rows_v.at[1], sg_b).wait()

            @pl.when(j + 2 < ch)
            def _():
                pltpu.async_copy(tab_hbm.at[idx_g_c.at[j + 2]], rows_v.at[0],
                                 sg_a)

            pltpu.sync_copy(rows_v.at[1], acc.at[idx_s_c.at[j + 1]],
                            add=True)


_SC_SCRATCH = [
    pltpu.VMEM((CH, B), jnp.int32),
    pltpu.VMEM((CH, B), jnp.int32),
    pltpu.VMEM((2, B, D), jnp.float32),
    pltpu.SemaphoreType.DMA,
    pltpu.SemaphoreType.DMA,
    pltpu.SemaphoreType.DMA,
    pltpu.SemaphoreType.DMA,
]


def _fused_body(w_c, w_g, w_gg, rev_g, rev_s, exp_g, exp_s, gg_g, gg_t,
                zeros, out_c_raw, out_ge_raw, out_gg_a, out_gg_b,
                acc, idx_g_c, idx_s_c, rows_v, sg_a, ss_a, sg_b, ss_b):
    core = lax.axis_index("core")
    sub = lax.axis_index("subcore")
    stripe = ACC_MAIN_ROWS // NS
    off = sub * stripe

    _zero_stripe(zeros, rows_v.at[0], acc, off, stripe)
    plsc.subcore_barrier()

    @pl.when(core == 0)
    def _():
        _run_relation(w_g, rev_g, rev_s, sub, 0, NB_MAIN, acc,
                      idx_g_c, idx_s_c, rows_v, sg_a, ss_a, sg_b, ss_b)

    @pl.when(core == 1)
    def _():
        _run_relation(w_c, exp_g, exp_s, sub, 0, NB_MAIN, acc,
                      idx_g_c, idx_s_c, rows_v, sg_a, ss_a, sg_b, ss_b)

    plsc.subcore_barrier()

    @pl.when(core == 0)
    def _():
        pltpu.sync_copy(acc.at[pl.ds(off, stripe)],
                        out_c_raw.at[pl.ds(off, stripe)])

    @pl.when(core == 1)
    def _():
        pltpu.sync_copy(acc.at[pl.ds(off, stripe)],
                        out_ge_raw.at[pl.ds(off, stripe)])

    _zero_stripe(zeros, rows_v.at[0], acc, off, stripe)
    plsc.subcore_barrier()

    @pl.when(core == 0)
    def _():
        _run_relation(w_gg, gg_g, gg_t, sub, 0, GG_B0, acc,
                      idx_g_c, idx_s_c, rows_v, sg_a, ss_a, sg_b, ss_b,
                      ch=8)

    @pl.when(core == 1)
    def _():
        _run_relation(w_gg, gg_g, gg_t, sub, GG_B0, NB_GG - GG_B0, acc,
                      idx_g_c, idx_s_c, rows_v, sg_a, ss_a, sg_b, ss_b,
                      ch=8)

    plsc.subcore_barrier()

    @pl.when(core == 0)
    def _():
        pltpu.sync_copy(acc.at[pl.ds(off, stripe)],
                        out_gg_a.at[pl.ds(off, stripe)])

    @pl.when(core == 1)
    def _():
        pltpu.sync_copy(acc.at[pl.ds(off, stripe)],
                        out_gg_b.at[pl.ds(off, stripe)])


def _sc_mesh():
    return plsc.VectorSubcoreMesh(core_axis_name="core",
                                  subcore_axis_name="subcore",
                                  num_cores=2, num_subcores=NS)


@functools.cache
def _fused_kernel():
    return pl.kernel(
        _fused_body,
        out_type=tuple(
            jax.ShapeDtypeStruct((ACC_MAIN_ROWS, D), jnp.float32)
            for _ in range(4)
        ),
        mesh=_sc_mesh(),
        scratch_types=[pltpu.VMEM_SHARED((ACC_MAIN_ROWS, D), jnp.float32)]
        + _SC_SCRATCH,
    )


def _prescale_body(c_ref, g_ref, cjc_ref, cjg_ref, cjj_ref,
                   wc_ref, wg_ref, wgg_ref):
    wc_ref[...] = c_ref[...] * cjc_ref[...]
    wg_ref[...] = g_ref[...] * cjg_ref[...]
    wgg_ref[...] = g_ref[...] * cjj_ref[...]


def _prescale(c_feat, g_feat, cj_cell, cj_gene, cjj_gene):
    blk = 1000
    grid = N_C // blk
    feat_spec = pl.BlockSpec((blk, D), lambda i: (i, 0))
    scale_spec = pl.BlockSpec((blk, 1), lambda i: (i, 0))
    return pl.pallas_call(
        _prescale_body,
        grid=(grid,),
        in_specs=[feat_spec, feat_spec, scale_spec, scale_spec, scale_spec],
        out_specs=[feat_spec, feat_spec, feat_spec],
        out_shape=[
            jax.ShapeDtypeStruct((N_C, D), jnp.float32),
            jax.ShapeDtypeStruct((N_G, D), jnp.float32),
            jax.ShapeDtypeStruct((N_G, D), jnp.float32),
        ],
    )(c_feat, g_feat, cj_cell, cj_gene, cjj_gene)


def _postscale_body(oc_ref, oge_ref, ga_ref, gb_ref, cic_ref, cig_ref,
                    cii_ref, out_c_ref, g_out_ref):
    out_c_ref[...] = oc_ref[...] * cic_ref[...]
    g_out_ref[...] = ALPHA1 * (oge_ref[...] * cig_ref[...]) + \
        (1.0 - ALPHA1) * ((ga_ref[...] + gb_ref[...]) * cii_ref[...])


def _postscale(oc_raw, oge_raw, ogg_a, ogg_b, ci_cell, ci_gene, cii_gene):
    blk = 1000
    grid = N_G // blk
    full_spec = pl.BlockSpec((blk, D), lambda i: (i, 0))
    scale_spec = pl.BlockSpec((blk, 1), lambda i: (i, 0))
    return pl.pallas_call(
        _postscale_body,
        grid=(grid,),
        in_specs=[full_spec] * 4 + [scale_spec] * 3,
        out_specs=[full_spec, full_spec],
        out_shape=[
            jax.ShapeDtypeStruct((N_C, D), jnp.float32),
            jax.ShapeDtypeStruct((N_G, D), jnp.float32),
        ],
    )(oc_raw, oge_raw, ogg_a, ogg_b, ci_cell, ci_gene, cii_gene)


def _pad_reshape(idx, total, pad_val, nt, nb):
    pad = total - idx.shape[0]
    idx = jnp.concatenate(
        [idx, jnp.full((pad,), pad_val, dtype=jnp.int32)])
    return idx.reshape(nt, nb, B)


def kernel(c_feat, g_feat, cj_cell, ci_cell, cj_gene, ci_gene, cjj_gene,
           cii_gene, exp_edge_index, gg_edge_index):
    exp_src = exp_edge_index[0].astype(jnp.int32)
    exp_dst = exp_edge_index[1].astype(jnp.int32)
    gg_s = gg_edge_index[0].astype(jnp.int32)
    gg_t = gg_edge_index[1].astype(jnp.int32)

    tot_main = NS * NB_MAIN * B
    tot_gg = NS * NB_GG * B
    rev_g = _pad_reshape(exp_dst, tot_main, 0, NS, NB_MAIN)
    rev_s = _pad_reshape(exp_src, tot_main, N_C, NS, NB_MAIN)
    exp_g = _pad_reshape(exp_src, tot_main, 0, NS, NB_MAIN)
    exp_s = _pad_reshape(exp_dst, tot_main, N_G, NS, NB_MAIN)
    gg_g = _pad_reshape(gg_s, tot_gg, 0, NS, NB_GG)
    gg_t2 = _pad_reshape(gg_t, tot_gg, N_G, NS, NB_GG)

    w_c, w_g, w_gg = _prescale(c_feat, g_feat, cj_cell, cj_gene, cjj_gene)
    zeros = jnp.zeros((B, D), jnp.float32)

    oc_raw, oge_raw, ogg_a, ogg_b = _fused_kernel()(
        w_c, w_g, w_gg, rev_g, rev_s, exp_g, exp_s, gg_g, gg_t2, zeros)

    out_c, g_out = _postscale(oc_raw, oge_raw, ogg_a, ogg_b,
                              ci_cell, ci_gene, cii_gene)
    return (out_c, g_out)
